# baseline (speedup 1.0000x reference)
import functools

import jax
import jax.numpy as jnp
from jax import lax
from jax.experimental import pallas as pl
from jax.experimental.pallas import tpu as pltpu
from jax.experimental.pallas import tpu_sc as plsc

VOCAB = 4608
HIDDEN = 384
HALF = HIDDEN // 2
PACKW = 256
EPS = 1e-6

NUM_CORES = 2
NUM_SUBCORES = 16
NUM_WORKERS = NUM_CORES * NUM_SUBCORES
LANES = 16


def _normalize_pack_body(table_ref, w_ref, out_ref):
    x = table_ref[...]
    var = jnp.mean(x * x, axis=-1, keepdims=True)
    y = x * lax.rsqrt(var + EPS) * w_ref[...][None, :]
    b = lax.bitcast_convert_type(y, jnp.uint32)
    code = (b + jnp.uint32(0x7FFF) + ((b >> 16) & jnp.uint32(1))) >> 16
    packed = (code[:, HALF:] << 16) | code[:, :HALF]
    out_ref[...] = jnp.pad(packed, ((0, 0), (0, PACKW - HALF))).astype(
        jnp.int32)


def _normalize_pack(table, w):
    blk = VOCAB // 4
    return pl.pallas_call(
        _normalize_pack_body,
        grid=(4,),
        in_specs=[
            pl.BlockSpec((blk, HIDDEN), lambda i: (i, 0)),
            pl.BlockSpec((HIDDEN,), lambda i: (0,)),
        ],
        out_specs=pl.BlockSpec((blk, PACKW), lambda i: (i, 0)),
        out_shape=jax.ShapeDtypeStruct((VOCAB, PACKW), jnp.int32),
    )(table, w)


def _make_gather(Brows, S, chunk, nbuf):
    B = Brows * S
    b_per_w = B // NUM_WORKERS
    w_per_row = NUM_WORKERS // Brows
    n_chunks = b_per_w // chunk
    mesh = plsc.VectorSubcoreMesh(
        core_axis_name="c", subcore_axis_name="s",
        num_cores=NUM_CORES, num_subcores=NUM_SUBCORES)

    @functools.partial(
        pl.kernel,
        mesh=mesh,
        out_type=jax.ShapeDtypeStruct((B, HIDDEN), jnp.float32),
        scratch_types=[
            pltpu.VMEM((b_per_w,), jnp.int32),
        ] + [pltpu.VMEM((chunk, PACKW), jnp.int32) for _ in range(nbuf)]
          + [pltpu.VMEM((chunk, HIDDEN), jnp.float32) for _ in range(nbuf)]
          + [
            pltpu.SemaphoreType.DMA,
            pltpu.SemaphoreType.DMA,
        ],
    )
    def gather_kernel(idx_hbm, ptab_hbm, out_hbm, idx_v, *rest):
        ibufs = rest[:nbuf]
        obufs = rest[nbuf:2 * nbuf]
        g_sem, s_sem = rest[2 * nbuf], rest[2 * nbuf + 1]
        wid = lax.axis_index("s") * NUM_CORES + lax.axis_index("c")
        base = wid * b_per_w
        pltpu.sync_copy(
            idx_hbm.at[wid // w_per_row,
                       pl.ds((wid % w_per_row) * b_per_w, b_per_w)],
            idx_v)

        def gather_start(c):
            pltpu.async_copy(
                ptab_hbm.at[idx_v.at[pl.ds(c * chunk, chunk)]],
                ibufs[c % nbuf], g_sem)

        def gather_wait(c):
            pltpu.make_async_copy(
                ptab_hbm.at[idx_v.at[pl.ds(c * chunk, chunk)]],
                ibufs[c % nbuf], g_sem).wait()

        def out_start(c):
            pltpu.async_copy(
                obufs[c % nbuf], out_hbm.at[pl.ds(base + c * chunk, chunk)],
                s_sem)

        def out_wait(c):
            pltpu.make_async_copy(
                obufs[c % nbuf], out_hbm.at[pl.ds(base + c * chunk, chunk)],
                s_sem).wait()

        def expand(c):
            ibuf = ibufs[c % nbuf]
            obuf = obufs[c % nbuf]

            @plsc.parallel_loop(0, chunk, unroll=2)
            def row(r):
                ws = [ibuf[r, pl.ds(k * LANES, LANES)]
                      for k in range(HALF // LANES)]
                for k, w32 in enumerate(ws):
                    lo = lax.bitcast_convert_type(w32 << 16, jnp.float32)
                    hi = lax.bitcast_convert_type(
                        w32 & jnp.int32(-65536), jnp.float32)
                    obuf[r, pl.ds(k * LANES, LANES)] = lo
                    obuf[r, pl.ds(HALF + k * LANES, LANES)] = hi

        for c in range(min(nbuf, n_chunks)):
            gather_start(c)
        for c in range(n_chunks):
            gather_wait(c)
            if c >= nbuf:
                out_wait(c - nbuf)
            expand(c)
            out_start(c)
            nxt = c + nbuf
            if nxt < n_chunks:
                gather_start(nxt)
        for c in range(max(0, n_chunks - nbuf), n_chunks):
            out_wait(c)

    return gather_kernel


def kernel(input_ids, tok_embeddings, norm_weight):
    B, S = input_ids.shape
    ids = input_ids.astype(jnp.int32)
    packed_table = _normalize_pack(tok_embeddings, norm_weight)
    gather = _make_gather(B, S, chunk=64, nbuf=3)
    out = gather(ids, packed_table)
    return out.reshape(B, S, HIDDEN)


# speedup vs baseline: 1.0097x; 1.0097x over previous
import functools

import jax
import jax.numpy as jnp
from jax import lax
from jax.experimental import pallas as pl
from jax.experimental.pallas import tpu as pltpu
from jax.experimental.pallas import tpu_sc as plsc

VOCAB = 4608
HIDDEN = 384
HALF = HIDDEN // 2
PACKW = 256
EPS = 1e-6

NUM_CORES = 2
NUM_SUBCORES = 16
NUM_WORKERS = NUM_CORES * NUM_SUBCORES
LANES = 16


def _normalize_pack_body(table_ref, w_ref, out_ref):
    x = table_ref[...]
    var = jnp.mean(x * x, axis=-1, keepdims=True)
    y = x * lax.rsqrt(var + EPS) * w_ref[...][None, :]
    b = lax.bitcast_convert_type(y, jnp.uint32)
    code = (b + jnp.uint32(0x7FFF) + ((b >> 16) & jnp.uint32(1))) >> 16
    packed = (code[:, HALF:] << 16) | code[:, :HALF]
    out_ref[...] = jnp.pad(packed, ((0, 0), (0, PACKW - HALF))).astype(
        jnp.int32)


def _normalize_pack(table, w):
    blk = VOCAB // 2
    return pl.pallas_call(
        _normalize_pack_body,
        grid=(2,),
        in_specs=[
            pl.BlockSpec((blk, HIDDEN), lambda i: (i, 0)),
            pl.BlockSpec((HIDDEN,), lambda i: (0,)),
        ],
        out_specs=pl.BlockSpec((blk, PACKW), lambda i: (i, 0)),
        out_shape=jax.ShapeDtypeStruct((VOCAB, PACKW), jnp.int32),
    )(table, w)


def _make_gather(Brows, S, chunk, nbuf):
    B = Brows * S
    b_per_w = B // NUM_WORKERS
    w_per_row = NUM_WORKERS // Brows
    n_chunks = b_per_w // chunk
    mesh = plsc.VectorSubcoreMesh(
        core_axis_name="c", subcore_axis_name="s",
        num_cores=NUM_CORES, num_subcores=NUM_SUBCORES)

    @functools.partial(
        pl.kernel,
        mesh=mesh,
        out_type=jax.ShapeDtypeStruct((B, HIDDEN), jnp.float32),
        scratch_types=[
            pltpu.VMEM((b_per_w,), jnp.int32),
        ] + [pltpu.VMEM((chunk, PACKW), jnp.int32) for _ in range(nbuf)]
          + [pltpu.VMEM((chunk, HIDDEN), jnp.float32) for _ in range(nbuf)]
          + [
            pltpu.SemaphoreType.DMA,
            pltpu.SemaphoreType.DMA,
        ],
    )
    def gather_kernel(idx_hbm, ptab_hbm, out_hbm, idx_v, *rest):
        ibufs = rest[:nbuf]
        obufs = rest[nbuf:2 * nbuf]
        g_sem, s_sem = rest[2 * nbuf], rest[2 * nbuf + 1]
        wid = lax.axis_index("s") * NUM_CORES + lax.axis_index("c")
        base = wid * b_per_w
        pltpu.sync_copy(
            idx_hbm.at[wid // w_per_row,
                       pl.ds((wid % w_per_row) * b_per_w, b_per_w)],
            idx_v)

        def gather_start(c):
            pltpu.async_copy(
                ptab_hbm.at[idx_v.at[pl.ds(c * chunk, chunk)]],
                ibufs[c % nbuf], g_sem)

        def gather_wait(c):
            pltpu.make_async_copy(
                ptab_hbm.at[idx_v.at[pl.ds(c * chunk, chunk)]],
                ibufs[c % nbuf], g_sem).wait()

        def out_start(c):
            pltpu.async_copy(
                obufs[c % nbuf], out_hbm.at[pl.ds(base + c * chunk, chunk)],
                s_sem)

        def out_wait(c):
            pltpu.make_async_copy(
                obufs[c % nbuf], out_hbm.at[pl.ds(base + c * chunk, chunk)],
                s_sem).wait()

        def expand(c):
            ibuf = ibufs[c % nbuf]
            obuf = obufs[c % nbuf]

            @plsc.parallel_loop(0, chunk, unroll=2)
            def row(r):
                ws = [ibuf[r, pl.ds(k * LANES, LANES)]
                      for k in range(HALF // LANES)]
                for k, w32 in enumerate(ws):
                    lo = lax.bitcast_convert_type(w32 << 16, jnp.float32)
                    hi = lax.bitcast_convert_type(
                        w32 & jnp.int32(-65536), jnp.float32)
                    obuf[r, pl.ds(k * LANES, LANES)] = lo
                    obuf[r, pl.ds(HALF + k * LANES, LANES)] = hi

        for c in range(min(nbuf, n_chunks)):
            gather_start(c)
        for c in range(n_chunks):
            gather_wait(c)
            if c >= nbuf:
                out_wait(c - nbuf)
            expand(c)
            out_start(c)
            nxt = c + nbuf
            if nxt < n_chunks:
                gather_start(nxt)
        for c in range(max(0, n_chunks - nbuf), n_chunks):
            out_wait(c)

    return gather_kernel


def kernel(input_ids, tok_embeddings, norm_weight):
    B, S = input_ids.shape
    ids = input_ids.astype(jnp.int32)
    packed_table = _normalize_pack(tok_embeddings, norm_weight)
    gather = _make_gather(B, S, chunk=64, nbuf=3)
    out = gather(ids, packed_table)
    return out.reshape(B, S, HIDDEN)
